# Initial kernel scaffold; baseline (speedup 1.0000x reference)
import jax
import jax.numpy as jnp
from jax.experimental import pallas as pl


def kernel(nodes, history_vc, history_r, c2e_weight, r2e_weight, v2e_weight, Wq, bq, Wk, bk, Wv, bv, Wo, bo):
    raise NotImplementedError("write your pallas kernel here")


# speedup vs baseline: 3.2087x; 3.2087x over previous
import functools

import jax
import jax.numpy as jnp
from jax import lax
from jax.experimental import pallas as pl
from jax.experimental.pallas import tpu as pltpu
from jax.experimental.pallas import tpu_sc as plsc

B = 4096
L = 50
D = 16
LP = 64
NC_PAD = 1024
NR_PAD = 16
NW = 32
ROWS = B // NW


def _tc_precompute(c2e_p, r2e_p, Wk, Wv, Wo, bv2, bo2):
    def body(c2e_ref, r2e_ref, wk_ref, wv_ref, wo_ref, bv_ref, bo_ref,
             skt_ref, sv_ref, rkt_ref, rvp_ref):
        c2e = c2e_ref[...]
        r2e = r2e_ref[...]
        wk0 = wk_ref[0:D, :]
        wk1 = wk_ref[D:2 * D, :]
        wv0 = wv_ref[0:D, :]
        wv1 = wv_ref[D:2 * D, :]
        wo = wo_ref[...]
        scale = 0.25
        sk = jnp.dot(c2e, wk0, preferred_element_type=jnp.float32) * scale
        skt_ref[...] = sk.T
        rk = jnp.dot(r2e, wk1, preferred_element_type=jnp.float32) * scale
        rkt_ref[...] = rk.T
        wvo0 = jnp.dot(wv0, wo, preferred_element_type=jnp.float32)
        wvo1 = jnp.dot(wv1, wo, preferred_element_type=jnp.float32)
        cb = jnp.dot(bv_ref[...], wo, preferred_element_type=jnp.float32) + bo_ref[...]
        sv_ref[...] = jnp.dot(c2e, wvo0, preferred_element_type=jnp.float32)
        rvp_ref[...] = jnp.dot(r2e, wvo1, preferred_element_type=jnp.float32) + cb

    return pl.pallas_call(
        body,
        out_shape=(
            jax.ShapeDtypeStruct((D, NC_PAD), jnp.float32),
            jax.ShapeDtypeStruct((NC_PAD, D), jnp.float32),
            jax.ShapeDtypeStruct((D, NR_PAD), jnp.float32),
            jax.ShapeDtypeStruct((NR_PAD, D), jnp.float32),
        ),
    )(c2e_p, r2e_p, Wk, Wv, Wo, bv2, bo2)


def _sc_attention(nodes, hvc_p, hr_p, v2e, Wq, bq2, skt, sv, rkt, rvp):
    mesh = plsc.VectorSubcoreMesh(core_axis_name="c", subcore_axis_name="s")

    @functools.partial(
        pl.kernel,
        mesh=mesh,
        compiler_params=pltpu.CompilerParams(
            needs_layout_passes=False, use_tc_tiling_on_sc=False),
        out_type=jax.ShapeDtypeStruct((B, D), jnp.float32),
        scratch_types=[
            pltpu.VMEM((D, NC_PAD), jnp.float32),
            pltpu.VMEM((NC_PAD, D), jnp.float32),
            pltpu.VMEM((D, NR_PAD), jnp.float32),
            pltpu.VMEM((NR_PAD, D), jnp.float32),
            pltpu.VMEM((D, D), jnp.float32),
            pltpu.VMEM((1, D), jnp.float32),
            pltpu.VMEM((ROWS,), jnp.int32),
            pltpu.VMEM((ROWS, D), jnp.float32),
            pltpu.VMEM((ROWS, LP), jnp.int32),
            pltpu.VMEM((ROWS, LP), jnp.int32),
            pltpu.VMEM((1, D), jnp.float32),
            pltpu.VMEM((ROWS, D), jnp.float32),
            pltpu.SemaphoreType.DMA,
        ],
    )
    def k(nodes_hbm, hvc_hbm, hr_hbm, v2e_hbm, wq_hbm, bq_hbm,
          skt_hbm, sv_hbm, rkt_hbm, rvp_hbm, out_hbm,
          skt_v, sv_v, rkt_v, rvp_v, wq_v, bq_v, nodes_v, vcrep_v,
          hvc_v, hr_v, rkq_buf, outbuf, sem):
        wid = lax.axis_index("c") * 16 + lax.axis_index("s")
        base = wid * ROWS

        pltpu.sync_copy(skt_hbm, skt_v)
        pltpu.sync_copy(sv_hbm, sv_v)
        pltpu.sync_copy(rkt_hbm, rkt_v)
        pltpu.sync_copy(rvp_hbm, rvp_v)
        pltpu.sync_copy(wq_hbm, wq_v)
        pltpu.sync_copy(bq_hbm, bq_v)
        pltpu.sync_copy(nodes_hbm.at[pl.ds(base, ROWS)], nodes_v)
        pltpu.sync_copy(hvc_hbm.at[pl.ds(base, ROWS)], hvc_v)
        pltpu.sync_copy(hr_hbm.at[pl.ds(base, ROWS)], hr_v)
        pltpu.async_copy(v2e_hbm.at[nodes_v], vcrep_v, sem).wait()

        iota = lax.iota(jnp.int32, 16)
        lanemask_last = iota < (L - 3 * 16)
        neg = jnp.full((16,), -1e30, jnp.float32)

        def row_body(i, carry):
            vcvec = vcrep_v[i, :]
            q = bq_v[0, :]
            for d in range(D):
                q = q + vcvec[d] * wq_v[d, :]
            qs = [q[d] for d in range(D)]
            rkq = jnp.zeros((16,), jnp.float32)
            for d in range(D):
                rkq = rkq + qs[d] * rkt_v[d, :]
            rkq_buf[0, :] = rkq
            chunks, cvs, rrs = [], [], []
            for t in range(LP // 16):
                cv = hvc_v[i, pl.ds(16 * t, 16)]
                rr = hr_v[i, pl.ds(16 * t, 16)]
                cvs.append(cv)
                rrs.append(rr)
                acc = plsc.load_gather(rkq_buf, [jnp.zeros((16,), jnp.int32), rr])
                for d in range(D):
                    dvec = jnp.full((16,), d, jnp.int32)
                    acc = acc + qs[d] * plsc.load_gather(skt_v, [dvec, cv])
                chunks.append(acc)
            chunks[3] = jnp.where(lanemask_last, chunks[3], neg)
            m = jnp.max(jnp.maximum(jnp.maximum(chunks[0], chunks[1]),
                                    jnp.maximum(chunks[2], chunks[3])))
            es = [jnp.exp(c - m) for c in chunks]
            total = jnp.sum(es[0] + es[1] + es[2] + es[3])
            inv = jnp.full((16,), 1.0, jnp.float32) / jnp.broadcast_to(total, (16,))
            out = jnp.zeros((16,), jnp.float32)
            for t in range(LP // 16):
                at = es[t] * inv
                for j in range(16):
                    l = 16 * t + j
                    if l >= L:
                        break
                    c = jnp.broadcast_to(cvs[t][j], (16,))
                    r = jnp.broadcast_to(rrs[t][j], (16,))
                    row = (plsc.load_gather(sv_v, [c, iota]) +
                           plsc.load_gather(rvp_v, [r, iota]))
                    out = out + at[j] * row
            plsc.store_scatter(outbuf, [jnp.broadcast_to(i, (16,)), iota], out)
            return carry

        lax.fori_loop(0, ROWS, row_body, 0)
        pltpu.sync_copy(outbuf, out_hbm.at[pl.ds(base, ROWS)])

    return k(nodes, hvc_p, hr_p, v2e, Wq, bq2, skt, sv, rkt, rvp)


def kernel(nodes, history_vc, history_r, c2e_weight, r2e_weight, v2e_weight,
           Wq, bq, Wk, bk, Wv, bv, Wo, bo):
    nodes = nodes.astype(jnp.int32)
    hvc_p = jnp.pad(history_vc.astype(jnp.int32), ((0, 0), (0, LP - L)))
    hr_p = jnp.pad(history_r.astype(jnp.int32), ((0, 0), (0, LP - L)))
    c2e_p = jnp.pad(c2e_weight, ((0, NC_PAD - c2e_weight.shape[0]), (0, 0)))
    r2e_p = jnp.pad(r2e_weight, ((0, NR_PAD - r2e_weight.shape[0]), (0, 0)))
    bv2 = bv.reshape(1, D)
    bo2 = bo.reshape(1, D)
    bq2 = bq.reshape(1, D)
    skt, sv, rkt, rvp = _tc_precompute(c2e_p, r2e_p, Wk, Wv, Wo, bv2, bo2)
    return _sc_attention(nodes, hvc_p, hr_p, v2e_weight, Wq, bq2,
                         skt, sv, rkt, rvp)
